# Initial kernel scaffold; baseline (speedup 1.0000x reference)
import jax
import jax.numpy as jnp
from jax.experimental import pallas as pl


def kernel(fmess_wid, mess_nei, root_wid, root_mess_nei, embedding, Wz_w, Wz_b, Wr_w, Ur_w, Ur_b, Wh_w, Wh_b, W_w, W_b):
    raise NotImplementedError("write your pallas kernel here")


# speedup vs baseline: 1.5127x; 1.5127x over previous
import functools

import jax
import jax.numpy as jnp
from jax import lax
from jax.experimental import pallas as pl
from jax.experimental.pallas import tpu as pltpu

H = 128
NB = 8
BM = 1000


def _sig(v):
    return 1.0 / (1.0 + jnp.exp(-v))


def _first_step_body(x_ref, wzt_ref, bz_ref, wht_ref, bh_ref, h_ref):
    x = x_ref[...]
    z = _sig(jnp.dot(x, wzt_ref[...], preferred_element_type=jnp.float32)
             + bz_ref[...])
    p = jnp.tanh(jnp.dot(x, wht_ref[...], preferred_element_type=jnp.float32)
                 + bh_ref[...])
    h_ref[...] = z * p


def _first_step(x, Wz_w, Wz_b, Wh_w, Wh_b):
    E = x.shape[0]
    grid = (E // BM,)
    return pl.pallas_call(
        _first_step_body,
        grid=grid,
        in_specs=[
            pl.BlockSpec((BM, H), lambda i: (i, 0)),
            pl.BlockSpec((H, H), lambda i: (0, 0)),
            pl.BlockSpec((1, H), lambda i: (0, 0)),
            pl.BlockSpec((H, H), lambda i: (0, 0)),
            pl.BlockSpec((1, H), lambda i: (0, 0)),
        ],
        out_specs=pl.BlockSpec((BM, H), lambda i: (i, 0)),
        out_shape=jax.ShapeDtypeStruct((E, H), jnp.float32),
    )(x, Wz_w[:H], Wz_b.reshape(1, H), Wh_w[:H], Wh_b.reshape(1, H))


def _gru_body(x_ref, hnei_ref, wz_ref, bz_ref, wr_ref, ur_ref, bu_ref,
              wh_ref, bh_ref, h_ref):
    x = x_ref[...]
    hnei = hnei_ref[...]
    sum_h = jnp.sum(hnei, axis=1)
    z = _sig(jnp.dot(x, wz_ref[0:H], preferred_element_type=jnp.float32)
             + jnp.dot(sum_h, wz_ref[H:2 * H],
                       preferred_element_type=jnp.float32)
             + bz_ref[...])
    r1 = jnp.dot(x, wr_ref[...], preferred_element_type=jnp.float32)
    r2 = lax.dot_general(hnei, ur_ref[...], (((2,), (0,)), ((), ())),
                         preferred_element_type=jnp.float32)
    r = _sig(r1[:, None, :] + r2 + bu_ref[...][None])
    sum_g = jnp.sum(r * hnei, axis=1)
    p = jnp.tanh(jnp.dot(x, wh_ref[0:H], preferred_element_type=jnp.float32)
                 + jnp.dot(sum_g, wh_ref[H:2 * H],
                           preferred_element_type=jnp.float32)
                 + bh_ref[...])
    h_ref[...] = (1.0 - z) * sum_h + z * p


def _gru_step(x, hnei, Wz_w, Wz_b, Wr_w, Ur_w, Ur_b, Wh_w, Wh_b):
    E = x.shape[0]
    grid = (E // BM,)
    return pl.pallas_call(
        _gru_body,
        grid=grid,
        in_specs=[
            pl.BlockSpec((BM, H), lambda i: (i, 0)),
            pl.BlockSpec((BM, NB, H), lambda i: (i, 0, 0)),
            pl.BlockSpec((2 * H, H), lambda i: (0, 0)),
            pl.BlockSpec((1, H), lambda i: (0, 0)),
            pl.BlockSpec((H, H), lambda i: (0, 0)),
            pl.BlockSpec((H, H), lambda i: (0, 0)),
            pl.BlockSpec((1, H), lambda i: (0, 0)),
            pl.BlockSpec((2 * H, H), lambda i: (0, 0)),
            pl.BlockSpec((1, H), lambda i: (0, 0)),
        ],
        out_specs=pl.BlockSpec((BM, H), lambda i: (i, 0)),
        out_shape=jax.ShapeDtypeStruct((E, H), jnp.float32),
    )(x, hnei, Wz_w, Wz_b.reshape(1, H), Wr_w, Ur_w, Ur_b.reshape(1, H),
      Wh_w, Wh_b.reshape(1, H))


def _root_body(xr_ref, hrnei_ref, w_ref, b_ref, out_ref):
    xr = xr_ref[...]
    s = jnp.sum(hrnei_ref[...], axis=1)
    v = (jnp.dot(xr, w_ref[0:H], preferred_element_type=jnp.float32)
         + jnp.dot(s, w_ref[H:2 * H], preferred_element_type=jnp.float32)
         + b_ref[...])
    out_ref[...] = jnp.maximum(v, 0.0)


def _root_agg(x_root, hrnei, W_w, W_b):
    R = x_root.shape[0]
    return pl.pallas_call(
        _root_body,
        grid=(1,),
        in_specs=[
            pl.BlockSpec((R, H), lambda i: (0, 0)),
            pl.BlockSpec((R, NB, H), lambda i: (0, 0, 0)),
            pl.BlockSpec((2 * H, H), lambda i: (0, 0)),
            pl.BlockSpec((1, H), lambda i: (0, 0)),
        ],
        out_specs=pl.BlockSpec((R, H), lambda i: (0, 0)),
        out_shape=jax.ShapeDtypeStruct((R, H), jnp.float32),
    )(x_root, hrnei, W_w, W_b.reshape(1, H))


def kernel(fmess_wid, mess_nei, root_wid, root_mess_nei, embedding,
           Wz_w, Wz_b, Wr_w, Ur_w, Ur_b, Wh_w, Wh_b, W_w, W_b):
    E = fmess_wid.shape[0]
    T = 6

    x = jnp.take(embedding, fmess_wid, axis=0)
    h = _first_step(x, Wz_w, Wz_b, Wh_w, Wh_b)
    flat_nei = mess_nei.reshape(-1)
    for _ in range(T - 1):
        hnei = jnp.take(h, flat_nei, axis=0).reshape(E, NB, H)
        h = _gru_step(x, hnei, Wz_w, Wz_b, Wr_w, Ur_w, Ur_b, Wh_w, Wh_b)

    R = root_wid.shape[0]
    x_root = jnp.take(embedding, root_wid, axis=0)
    hrnei = jnp.take(h, root_mess_nei.reshape(-1), axis=0).reshape(R, NB, H)
    root_vecs = _root_agg(x_root, hrnei, W_w, W_b)
    return (h, root_vecs)


# speedup vs baseline: 3.5691x; 2.3594x over previous
import functools

import jax
import jax.numpy as jnp
from jax import lax
from jax.experimental import pallas as pl
from jax.experimental.pallas import tpu as pltpu
from jax.experimental.pallas import tpu_sc as plsc

H = 128
NB = 8
BM = 1000
CH = 128
NW = 32


def _sc_gather_body(n_chunks, per_w, table_hbm, idx_hbm, out_hbm,
                    idx_v, rows_v, sem):
    wid = lax.axis_index("s") * 2 + lax.axis_index("c")

    def body(i, carry):
        c = wid + i * NW

        @pl.when(c < n_chunks)
        def _():
            base = c * CH
            pltpu.sync_copy(idx_hbm.at[pl.ds(base, CH)], idx_v)
            pltpu.async_copy(table_hbm.at[idx_v], rows_v, sem).wait()
            pltpu.sync_copy(rows_v, out_hbm.at[pl.ds(base, CH)])

        return carry

    lax.fori_loop(0, per_w, body, 0)


def _sc_gather(table, idx):
    B = idx.shape[0]
    n_chunks = B // CH
    assert n_chunks * CH == B
    per_w = -(-n_chunks // NW)
    mesh = plsc.VectorSubcoreMesh(core_axis_name="c", subcore_axis_name="s")
    run = functools.partial(
        pl.kernel,
        mesh=mesh,
        out_type=jax.ShapeDtypeStruct((B, H), jnp.float32),
        scratch_types=[
            pltpu.VMEM((CH,), jnp.int32),
            pltpu.VMEM((CH, H), jnp.float32),
            pltpu.SemaphoreType.DMA,
        ],
    )(functools.partial(_sc_gather_body, n_chunks, per_w))
    return run(table, idx)


def _sig(v):
    return 1.0 / (1.0 + jnp.exp(-v))


def _first_step_body(x_ref, wzt_ref, bz_ref, wht_ref, bh_ref, h_ref):
    x = x_ref[...]
    z = _sig(jnp.dot(x, wzt_ref[...], preferred_element_type=jnp.float32)
             + bz_ref[...])
    p = jnp.tanh(jnp.dot(x, wht_ref[...], preferred_element_type=jnp.float32)
                 + bh_ref[...])
    h_ref[...] = z * p


def _first_step(x, Wz_w, Wz_b, Wh_w, Wh_b):
    E = x.shape[0]
    grid = (E // BM,)
    return pl.pallas_call(
        _first_step_body,
        grid=grid,
        in_specs=[
            pl.BlockSpec((BM, H), lambda i: (i, 0)),
            pl.BlockSpec((H, H), lambda i: (0, 0)),
            pl.BlockSpec((1, H), lambda i: (0, 0)),
            pl.BlockSpec((H, H), lambda i: (0, 0)),
            pl.BlockSpec((1, H), lambda i: (0, 0)),
        ],
        out_specs=pl.BlockSpec((BM, H), lambda i: (i, 0)),
        out_shape=jax.ShapeDtypeStruct((E, H), jnp.float32),
    )(x, Wz_w[:H], Wz_b.reshape(1, H), Wh_w[:H], Wh_b.reshape(1, H))


def _gru_body(x_ref, hnei_ref, wz_ref, bz_ref, wr_ref, ur_ref, bu_ref,
              wh_ref, bh_ref, h_ref):
    x = x_ref[...]
    hnei = hnei_ref[...]
    sum_h = jnp.sum(hnei, axis=1)
    z = _sig(jnp.dot(x, wz_ref[0:H], preferred_element_type=jnp.float32)
             + jnp.dot(sum_h, wz_ref[H:2 * H],
                       preferred_element_type=jnp.float32)
             + bz_ref[...])
    r1 = jnp.dot(x, wr_ref[...], preferred_element_type=jnp.float32)
    r2 = lax.dot_general(hnei, ur_ref[...], (((2,), (0,)), ((), ())),
                         preferred_element_type=jnp.float32)
    r = _sig(r1[:, None, :] + r2 + bu_ref[...][None])
    sum_g = jnp.sum(r * hnei, axis=1)
    p = jnp.tanh(jnp.dot(x, wh_ref[0:H], preferred_element_type=jnp.float32)
                 + jnp.dot(sum_g, wh_ref[H:2 * H],
                           preferred_element_type=jnp.float32)
                 + bh_ref[...])
    h_ref[...] = (1.0 - z) * sum_h + z * p


def _gru_step(x, hnei, Wz_w, Wz_b, Wr_w, Ur_w, Ur_b, Wh_w, Wh_b):
    E = x.shape[0]
    grid = (E // BM,)
    return pl.pallas_call(
        _gru_body,
        grid=grid,
        in_specs=[
            pl.BlockSpec((BM, H), lambda i: (i, 0)),
            pl.BlockSpec((BM, NB, H), lambda i: (i, 0, 0)),
            pl.BlockSpec((2 * H, H), lambda i: (0, 0)),
            pl.BlockSpec((1, H), lambda i: (0, 0)),
            pl.BlockSpec((H, H), lambda i: (0, 0)),
            pl.BlockSpec((H, H), lambda i: (0, 0)),
            pl.BlockSpec((1, H), lambda i: (0, 0)),
            pl.BlockSpec((2 * H, H), lambda i: (0, 0)),
            pl.BlockSpec((1, H), lambda i: (0, 0)),
        ],
        out_specs=pl.BlockSpec((BM, H), lambda i: (i, 0)),
        out_shape=jax.ShapeDtypeStruct((E, H), jnp.float32),
    )(x, hnei, Wz_w, Wz_b.reshape(1, H), Wr_w, Ur_w, Ur_b.reshape(1, H),
      Wh_w, Wh_b.reshape(1, H))


def _root_body(xr_ref, hrnei_ref, w_ref, b_ref, out_ref):
    xr = xr_ref[...]
    s = jnp.sum(hrnei_ref[...], axis=1)
    v = (jnp.dot(xr, w_ref[0:H], preferred_element_type=jnp.float32)
         + jnp.dot(s, w_ref[H:2 * H], preferred_element_type=jnp.float32)
         + b_ref[...])
    out_ref[...] = jnp.maximum(v, 0.0)


def _root_agg(x_root, hrnei, W_w, W_b):
    R = x_root.shape[0]
    return pl.pallas_call(
        _root_body,
        grid=(1,),
        in_specs=[
            pl.BlockSpec((R, H), lambda i: (0, 0)),
            pl.BlockSpec((R, NB, H), lambda i: (0, 0, 0)),
            pl.BlockSpec((2 * H, H), lambda i: (0, 0)),
            pl.BlockSpec((1, H), lambda i: (0, 0)),
        ],
        out_specs=pl.BlockSpec((R, H), lambda i: (0, 0)),
        out_shape=jax.ShapeDtypeStruct((R, H), jnp.float32),
    )(x_root, hrnei, W_w, W_b.reshape(1, H))


def kernel(fmess_wid, mess_nei, root_wid, root_mess_nei, embedding,
           Wz_w, Wz_b, Wr_w, Ur_w, Ur_b, Wh_w, Wh_b, W_w, W_b):
    E = fmess_wid.shape[0]
    T = 6

    def pad_to_chunks(a):
        rem = (-a.shape[0]) % CH
        return jnp.pad(a, (0, rem)) if rem else a

    x = _sc_gather(embedding, pad_to_chunks(fmess_wid))[:E]
    h = _first_step(x, Wz_w, Wz_b, Wh_w, Wh_b)
    flat_nei = mess_nei.reshape(-1)
    for _ in range(T - 1):
        hnei = _sc_gather(h, flat_nei).reshape(E, NB, H)
        h = _gru_step(x, hnei, Wz_w, Wz_b, Wr_w, Ur_w, Ur_b, Wh_w, Wh_b)

    R = root_wid.shape[0]
    x_root = _sc_gather(embedding, pad_to_chunks(root_wid))[:R]
    hrnei = _sc_gather(h, root_mess_nei.reshape(-1)).reshape(R, NB, H)
    root_vecs = _root_agg(x_root, hrnei, W_w, W_b)
    return (h, root_vecs)
